# Initial kernel scaffold; baseline (speedup 1.0000x reference)
import jax
import jax.numpy as jnp
from jax.experimental import pallas as pl


def kernel(x, embedding):
    raise NotImplementedError("write your pallas kernel here")


# speedup vs baseline: 9.8466x; 9.8466x over previous
import functools

import jax
import jax.numpy as jnp
from jax import lax
from jax.experimental import pallas as pl
from jax.experimental.pallas import tpu as pltpu
from jax.experimental.pallas import tpu_sc as plsc

D = 256
K = 8192
BN = 256
SC_CHUNK = 128


def _dist_kernel(x_ref, e_ref, idx_ref, md_ref, loss_ref, acc_ref):
    i = pl.program_id(0)
    ng = pl.num_programs(0)
    x = x_ref[...]
    e = e_ref[...]
    xe = lax.dot_general(x, e, (((1,), (0,)), ((), ())),
                         preferred_element_type=jnp.float32)
    e2 = jnp.sum(e * e, axis=0, keepdims=True)
    dist = e2 - 2.0 * xe
    minv = jnp.min(dist, axis=1, keepdims=True)
    kio = lax.broadcasted_iota(jnp.int32, dist.shape, 1)
    idx = jnp.min(jnp.where(dist == minv, kio, K), axis=1)
    idx_ref[0, 0, :] = idx
    x2 = jnp.sum(x * x, axis=1)
    md = minv[:, 0] + x2
    md_ref[0, 0, :] = md

    @pl.when(i == 0)
    def _():
        acc_ref[...] = jnp.zeros_like(acc_ref)

    acc_ref[...] += md[None, :]

    @pl.when(i == ng - 1)
    def _():
        loss_ref[...] = jnp.sum(acc_ref[...], keepdims=True).reshape(1, 1)


def _make_sc_gather(n, d):
    info = plsc.get_sparse_core_info()
    nc, ns = info.num_cores, info.num_subcores
    nw = nc * ns
    per_w = n // nw
    n_ch = per_w // SC_CHUNK
    mesh = plsc.VectorSubcoreMesh(core_axis_name="c", subcore_axis_name="s")

    @functools.partial(
        pl.kernel, mesh=mesh,
        out_type=jax.ShapeDtypeStruct((n, d), jnp.float32),
        scratch_types=[
            pltpu.VMEM((per_w,), jnp.int32),
            pltpu.VMEM((SC_CHUNK, d), jnp.float32),
            pltpu.SemaphoreType.DMA,
        ],
    )
    def gather_kernel(table_hbm, idx_hbm, out_hbm, idx_v, rows_v, sem):
        wid = lax.axis_index("s") * nc + lax.axis_index("c")
        pltpu.sync_copy(idx_hbm.at[pl.ds(wid * per_w, per_w)], idx_v)
        for c in range(n_ch):
            pltpu.async_copy(
                table_hbm.at[idx_v.at[pl.ds(c * SC_CHUNK, SC_CHUNK)]],
                rows_v, sem).wait()
            pltpu.sync_copy(rows_v,
                            out_hbm.at[pl.ds((wid * n_ch + c) * SC_CHUNK, SC_CHUNK)])

    return gather_kernel


def kernel(x, embedding):
    n = x.shape[0]
    g = n // BN
    idx3, _md3, loss_sum = pl.pallas_call(
        _dist_kernel,
        grid=(g,),
        in_specs=[
            pl.BlockSpec((BN, D), lambda i: (i, 0)),
            pl.BlockSpec((D, K), lambda i: (0, 0)),
        ],
        out_specs=[
            pl.BlockSpec((1, 1, BN), lambda i: (i, 0, 0)),
            pl.BlockSpec((1, 1, BN), lambda i: (i, 0, 0)),
            pl.BlockSpec((1, 1), lambda i: (0, 0)),
        ],
        out_shape=[
            jax.ShapeDtypeStruct((g, 1, BN), jnp.int32),
            jax.ShapeDtypeStruct((g, 1, BN), jnp.float32),
            jax.ShapeDtypeStruct((1, 1), jnp.float32),
        ],
        scratch_shapes=[pltpu.VMEM((1, BN), jnp.float32)],
    )(x, embedding)

    idx = idx3.reshape(n)
    loss = loss_sum[0, 0] * (0.25 / (n * D))
    table = embedding.T
    quantized = _make_sc_gather(n, D)(table, idx)
    return quantized, loss, idx.reshape(n, 1)


# speedup vs baseline: 11.0397x; 1.1212x over previous
import functools

import jax
import jax.numpy as jnp
from jax import lax
from jax.experimental import pallas as pl
from jax.experimental.pallas import tpu as pltpu
from jax.experimental.pallas import tpu_sc as plsc

D = 256
K = 8192
BN = 512
KT = 2048
SC_CHUNK = 128


def _dist_kernel(x_ref, e_ref, idx_ref, loss_ref, e2_ref, acc_ref):
    i = pl.program_id(0)
    ng = pl.num_programs(0)

    @pl.when(i == 0)
    def _():
        e2_ref[...] = jnp.sum(e_ref[...] * e_ref[...], axis=0, keepdims=True)
        acc_ref[...] = jnp.zeros_like(acc_ref)

    xm2 = x_ref[...] * -2.0
    best_v = None
    best_i = None
    for t in range(K // KT):
        e = e_ref[:, t * KT:(t + 1) * KT]
        xe = lax.dot_general(xm2, e, (((1,), (0,)), ((), ())),
                             preferred_element_type=jnp.float32)
        dist = xe + e2_ref[0, t * KT:(t + 1) * KT][None, :]
        minv = jnp.min(dist, axis=1, keepdims=True)
        kio = lax.broadcasted_iota(jnp.int32, dist.shape, 1) + t * KT
        tidx = jnp.min(jnp.where(dist == minv, kio, K), axis=1, keepdims=True)
        if t == 0:
            best_v, best_i = minv, tidx
        else:
            upd = minv < best_v
            best_i = jnp.where(upd, tidx, best_i)
            best_v = jnp.where(upd, minv, best_v)

    idx_ref[0, 0, :] = best_i[:, 0]
    x2 = jnp.sum(x_ref[...] * x_ref[...], axis=1)
    acc_ref[...] += (best_v[:, 0] + x2)[None, :]

    @pl.when(i == ng - 1)
    def _():
        loss_ref[...] = jnp.sum(acc_ref[...], keepdims=True).reshape(1, 1)


def _make_sc_gather(n, d):
    info = plsc.get_sparse_core_info()
    nc, ns = info.num_cores, info.num_subcores
    nw = nc * ns
    per_w = n // nw
    n_ch = per_w // SC_CHUNK
    mesh = plsc.VectorSubcoreMesh(core_axis_name="c", subcore_axis_name="s")

    @functools.partial(
        pl.kernel, mesh=mesh,
        out_type=jax.ShapeDtypeStruct((n, d), jnp.float32),
        scratch_types=[
            pltpu.VMEM((per_w,), jnp.int32),
            pltpu.VMEM((SC_CHUNK, d), jnp.float32),
            pltpu.SemaphoreType.DMA,
        ],
    )
    def gather_kernel(table_hbm, idx_hbm, out_hbm, idx_v, rows_v, sem):
        wid = lax.axis_index("s") * nc + lax.axis_index("c")
        pltpu.sync_copy(idx_hbm.at[pl.ds(wid * per_w, per_w)], idx_v)
        for c in range(n_ch):
            pltpu.async_copy(
                table_hbm.at[idx_v.at[pl.ds(c * SC_CHUNK, SC_CHUNK)]],
                rows_v, sem).wait()
            pltpu.sync_copy(rows_v,
                            out_hbm.at[pl.ds((wid * n_ch + c) * SC_CHUNK, SC_CHUNK)])

    return gather_kernel


def kernel(x, embedding):
    n = x.shape[0]
    g = n // BN
    idx3, loss_sum = pl.pallas_call(
        _dist_kernel,
        grid=(g,),
        in_specs=[
            pl.BlockSpec((BN, D), lambda i: (i, 0)),
            pl.BlockSpec((D, K), lambda i: (0, 0)),
        ],
        out_specs=[
            pl.BlockSpec((1, 1, BN), lambda i: (i, 0, 0)),
            pl.BlockSpec((1, 1), lambda i: (0, 0)),
        ],
        out_shape=[
            jax.ShapeDtypeStruct((g, 1, BN), jnp.int32),
            jax.ShapeDtypeStruct((1, 1), jnp.float32),
        ],
        scratch_shapes=[
            pltpu.VMEM((1, K), jnp.float32),
            pltpu.VMEM((1, BN), jnp.float32),
        ],
    )(x, embedding)

    idx = idx3.reshape(n)
    loss = loss_sum[0, 0] * (0.25 / (n * D))
    table = embedding.T
    quantized = _make_sc_gather(n, D)(table, idx)
    return quantized, loss, idx.reshape(n, 1)


# speedup vs baseline: 15.8550x; 1.4362x over previous
import functools

import jax
import jax.numpy as jnp
from jax import lax
from jax.experimental import pallas as pl
from jax.experimental.pallas import tpu as pltpu
from jax.experimental.pallas import tpu_sc as plsc

D = 256
K = 8192
BN = 512
KT = 2048
SC_CHUNK = 128


def _dist_kernel(x_ref, e_ref, idx_ref, loss_ref, e2_ref, acc_ref):
    i = pl.program_id(0)
    ng = pl.num_programs(0)

    @pl.when(i == 0)
    def _():
        e2_ref[...] = jnp.sum(e_ref[...] * e_ref[...], axis=0, keepdims=True)
        acc_ref[...] = jnp.zeros_like(acc_ref)

    xm2 = x_ref[...] * -2.0
    lane = lax.broadcasted_iota(jnp.int32, (BN, 128), 1)
    best = None
    bg = None
    for t in range(K // KT):
        e = e_ref[:, t * KT:(t + 1) * KT]
        xe = lax.dot_general(xm2, e, (((1,), (0,)), ((), ())),
                             preferred_element_type=jnp.float32)
        for j in range(KT // 128):
            g = t * (KT // 128) + j
            d = (xe[:, j * 128:(j + 1) * 128]
                 + e2_ref[0, g * 128:(g + 1) * 128][None, :])
            gi = jnp.full((BN, 128), g, jnp.int32)
            if best is None:
                best, bg = d, gi
            else:
                m = d < best
                best = jnp.where(m, d, best)
                bg = jnp.where(m, gi, bg)

    col = bg * 128 + lane
    minv = jnp.min(best, axis=1, keepdims=True)
    idx = jnp.min(jnp.where(best == minv, col, K), axis=1)
    idx_ref[0, 0, :] = idx
    x2 = jnp.sum(x_ref[...] * x_ref[...], axis=1)
    acc_ref[...] += (minv[:, 0] + x2)[None, :]

    @pl.when(i == ng - 1)
    def _():
        loss_ref[...] = jnp.sum(acc_ref[...], keepdims=True).reshape(1, 1)


def _make_sc_gather(n, d):
    info = plsc.get_sparse_core_info()
    nc, ns = info.num_cores, info.num_subcores
    nw = nc * ns
    per_w = n // nw
    n_ch = per_w // SC_CHUNK
    mesh = plsc.VectorSubcoreMesh(core_axis_name="c", subcore_axis_name="s")

    @functools.partial(
        pl.kernel, mesh=mesh,
        out_type=jax.ShapeDtypeStruct((n, d), jnp.float32),
        scratch_types=[
            pltpu.VMEM((per_w,), jnp.int32),
            pltpu.VMEM((SC_CHUNK, d), jnp.float32),
            pltpu.SemaphoreType.DMA,
        ],
    )
    def gather_kernel(table_hbm, idx_hbm, out_hbm, idx_v, rows_v, sem):
        wid = lax.axis_index("s") * nc + lax.axis_index("c")
        pltpu.sync_copy(idx_hbm.at[pl.ds(wid * per_w, per_w)], idx_v)
        for c in range(n_ch):
            pltpu.async_copy(
                table_hbm.at[idx_v.at[pl.ds(c * SC_CHUNK, SC_CHUNK)]],
                rows_v, sem).wait()
            pltpu.sync_copy(rows_v,
                            out_hbm.at[pl.ds((wid * n_ch + c) * SC_CHUNK, SC_CHUNK)])

    return gather_kernel


def kernel(x, embedding):
    n = x.shape[0]
    g = n // BN
    idx3, loss_sum = pl.pallas_call(
        _dist_kernel,
        grid=(g,),
        in_specs=[
            pl.BlockSpec((BN, D), lambda i: (i, 0)),
            pl.BlockSpec((D, K), lambda i: (0, 0)),
        ],
        out_specs=[
            pl.BlockSpec((1, 1, BN), lambda i: (i, 0, 0)),
            pl.BlockSpec((1, 1), lambda i: (0, 0)),
        ],
        out_shape=[
            jax.ShapeDtypeStruct((g, 1, BN), jnp.int32),
            jax.ShapeDtypeStruct((1, 1), jnp.float32),
        ],
        scratch_shapes=[
            pltpu.VMEM((1, K), jnp.float32),
            pltpu.VMEM((1, BN), jnp.float32),
        ],
    )(x, embedding)

    idx = idx3.reshape(n)
    loss = loss_sum[0, 0] * (0.25 / (n * D))
    table = embedding.T
    quantized = _make_sc_gather(n, D)(table, idx)
    return quantized, loss, idx.reshape(n, 1)


# speedup vs baseline: 17.5502x; 1.1069x over previous
import functools

import jax
import jax.numpy as jnp
from jax import lax
from jax.experimental import pallas as pl
from jax.experimental.pallas import tpu as pltpu
from jax.experimental.pallas import tpu_sc as plsc

D = 256
K = 8192
BN = 512
KT = 2048
SC_CHUNK = 128


def _dist_kernel(x_ref, e_ref, idx_ref, loss_ref, e2_ref, acc_ref):
    i = pl.program_id(0)
    ng = pl.num_programs(0)

    @pl.when(i == 0)
    def _():
        e2_ref[...] = jnp.sum(e_ref[...] * e_ref[...], axis=0, keepdims=True)
        acc_ref[...] = jnp.zeros_like(acc_ref)

    xm2 = x_ref[...] * -2.0
    lane = lax.broadcasted_iota(jnp.int32, (BN, 128), 1)
    best = None
    bg = None
    for t in range(K // KT):
        e = e_ref[:, t * KT:(t + 1) * KT]
        xe = lax.dot_general(xm2, e, (((1,), (0,)), ((), ())),
                             preferred_element_type=jnp.float32)
        for j in range(KT // 128):
            g = t * (KT // 128) + j
            d = (xe[:, j * 128:(j + 1) * 128]
                 + e2_ref[0, g * 128:(g + 1) * 128][None, :])
            gi = jnp.full((BN, 128), g, jnp.int32)
            if best is None:
                best, bg = d, gi
            else:
                m = d < best
                best = jnp.where(m, d, best)
                bg = jnp.where(m, gi, bg)

    col = bg * 128 + lane
    minv = jnp.min(best, axis=1, keepdims=True)
    idx = jnp.min(jnp.where(best == minv, col, K), axis=1)
    idx_ref[0, 0, :] = idx
    x2 = jnp.sum(x_ref[...] * x_ref[...], axis=1, keepdims=True)
    acc_ref[...] += minv + x2

    @pl.when(i == ng - 1)
    def _():
        loss_ref[...] = jnp.sum(acc_ref[...], keepdims=True).reshape(1, 1)


def _make_sc_gather(n, d):
    info = plsc.get_sparse_core_info()
    nc, ns = info.num_cores, info.num_subcores
    nw = nc * ns
    per_w = n // nw
    n_ch = per_w // SC_CHUNK
    mesh = plsc.VectorSubcoreMesh(core_axis_name="c", subcore_axis_name="s")

    @functools.partial(
        pl.kernel, mesh=mesh,
        out_type=jax.ShapeDtypeStruct((n, d), jnp.float32),
        scratch_types=[
            pltpu.VMEM((per_w,), jnp.int32),
            pltpu.VMEM((SC_CHUNK, d), jnp.float32),
            pltpu.SemaphoreType.DMA,
        ],
    )
    def gather_kernel(table_hbm, idx_hbm, out_hbm, idx_v, rows_v, sem):
        wid = lax.axis_index("s") * nc + lax.axis_index("c")
        pltpu.sync_copy(idx_hbm.at[pl.ds(wid * per_w, per_w)], idx_v)
        for c in range(n_ch):
            pltpu.async_copy(
                table_hbm.at[idx_v.at[pl.ds(c * SC_CHUNK, SC_CHUNK)]],
                rows_v, sem).wait()
            pltpu.sync_copy(rows_v,
                            out_hbm.at[pl.ds((wid * n_ch + c) * SC_CHUNK, SC_CHUNK)])

    return gather_kernel


def kernel(x, embedding):
    n = x.shape[0]
    g = n // BN
    idx3, loss_sum = pl.pallas_call(
        _dist_kernel,
        grid=(g,),
        in_specs=[
            pl.BlockSpec((BN, D), lambda i: (i, 0)),
            pl.BlockSpec((D, K), lambda i: (0, 0)),
        ],
        out_specs=[
            pl.BlockSpec((1, 1, BN), lambda i: (i, 0, 0)),
            pl.BlockSpec((1, 1), lambda i: (0, 0)),
        ],
        out_shape=[
            jax.ShapeDtypeStruct((g, 1, BN), jnp.int32),
            jax.ShapeDtypeStruct((1, 1), jnp.float32),
        ],
        scratch_shapes=[
            pltpu.VMEM((1, K), jnp.float32),
            pltpu.VMEM((BN, 1), jnp.float32),
        ],
    )(x, embedding)

    idx = idx3.reshape(n)
    loss = loss_sum[0, 0] * (0.25 / (n * D))
    table = embedding.T
    quantized = _make_sc_gather(n, D)(table, idx)
    return quantized, loss, idx.reshape(n, 1)


# speedup vs baseline: 17.6121x; 1.0035x over previous
import functools

import jax
import jax.numpy as jnp
from jax import lax
from jax.experimental import pallas as pl
from jax.experimental.pallas import tpu as pltpu
from jax.experimental.pallas import tpu_sc as plsc

D = 256
K = 8192
BN = 512
KT = 2048
SC_CHUNK = 128


def _dist_kernel(x_ref, e_ref, idx_ref, loss_ref, e2_ref, acc_ref):
    i = pl.program_id(0)
    ng = pl.num_programs(0)

    @pl.when(i == 0)
    def _():
        e2_ref[...] = jnp.sum(e_ref[...] * e_ref[...], axis=0, keepdims=True)
        acc_ref[...] = jnp.zeros_like(acc_ref)

    xm2 = x_ref[...] * -2.0
    lane = lax.broadcasted_iota(jnp.int32, (BN, 128), 1)
    best = None
    bg = None
    for t in range(K // KT):
        e = e_ref[:, t * KT:(t + 1) * KT]
        xe = lax.dot_general(xm2, e, (((1,), (0,)), ((), ())),
                             preferred_element_type=jnp.float32)
        for j in range(KT // 128):
            g = t * (KT // 128) + j
            d = (xe[:, j * 128:(j + 1) * 128]
                 + e2_ref[0, g * 128:(g + 1) * 128][None, :])
            gi = jnp.full((BN, 128), g, jnp.int32)
            if best is None:
                best, bg = d, gi
            else:
                m = d < best
                best = jnp.where(m, d, best)
                bg = jnp.where(m, gi, bg)

    col = bg * 128 + lane
    minv = jnp.min(best, axis=1, keepdims=True)
    idx = jnp.min(jnp.where(best == minv, col, K), axis=1)
    idx_ref[0, 0, :] = idx
    x2 = jnp.sum(x_ref[...] * x_ref[...], axis=1, keepdims=True)
    acc_ref[...] += minv + x2

    @pl.when(i == ng - 1)
    def _():
        loss_ref[...] = jnp.sum(acc_ref[...], keepdims=True).reshape(1, 1)


def _make_sc_gather(n, d):
    info = plsc.get_sparse_core_info()
    nc, ns = info.num_cores, info.num_subcores
    nw = nc * ns
    per_w = n // nw
    n_ch = per_w // SC_CHUNK
    mesh = plsc.VectorSubcoreMesh(core_axis_name="c", subcore_axis_name="s")

    @functools.partial(
        pl.kernel, mesh=mesh,
        out_type=jax.ShapeDtypeStruct((n, d), jnp.float32),
        scratch_types=[
            pltpu.VMEM((per_w,), jnp.int32),
            pltpu.VMEM((SC_CHUNK, d), jnp.float32),
            pltpu.VMEM((SC_CHUNK, d), jnp.float32),
            pltpu.SemaphoreType.DMA,
            pltpu.SemaphoreType.DMA,
        ],
    )
    def gather_kernel(table_hbm, idx_hbm, out_hbm, idx_v, rows0, rows1,
                      sem0, sem1):
        wid = lax.axis_index("s") * nc + lax.axis_index("c")
        pltpu.sync_copy(idx_hbm.at[pl.ds(wid * per_w, per_w)], idx_v)
        rows = (rows0, rows1)
        sems = (sem0, sem1)

        def fire(c):
            return pltpu.async_copy(
                table_hbm.at[idx_v.at[pl.ds(c * SC_CHUNK, SC_CHUNK)]],
                rows[c % 2], sems[c % 2])

        gcp = [fire(0), fire(1)]
        for c in range(n_ch):
            b = c % 2
            gcp[b].wait()
            pltpu.sync_copy(rows[b],
                            out_hbm.at[pl.ds((wid * n_ch + c) * SC_CHUNK, SC_CHUNK)])
            if c + 2 < n_ch:
                gcp[b] = fire(c + 2)

    return gather_kernel


def kernel(x, embedding):
    n = x.shape[0]
    g = n // BN
    idx3, loss_sum = pl.pallas_call(
        _dist_kernel,
        grid=(g,),
        in_specs=[
            pl.BlockSpec((BN, D), lambda i: (i, 0)),
            pl.BlockSpec((D, K), lambda i: (0, 0)),
        ],
        out_specs=[
            pl.BlockSpec((1, 1, BN), lambda i: (i, 0, 0)),
            pl.BlockSpec((1, 1), lambda i: (0, 0)),
        ],
        out_shape=[
            jax.ShapeDtypeStruct((g, 1, BN), jnp.int32),
            jax.ShapeDtypeStruct((1, 1), jnp.float32),
        ],
        scratch_shapes=[
            pltpu.VMEM((1, K), jnp.float32),
            pltpu.VMEM((BN, 1), jnp.float32),
        ],
    )(x, embedding)

    idx = idx3.reshape(n)
    loss = loss_sum[0, 0] * (0.25 / (n * D))
    table = embedding.T
    quantized = _make_sc_gather(n, D)(table, idx)
    return quantized, loss, idx.reshape(n, 1)
